# baseline (device time: 65849 ns/iter reference)
import jax
import jax.numpy as jnp
from jax import lax
from jax.experimental import pallas as pl
from jax.experimental.pallas import tpu as pltpu

N_DEV = 4
SQ = 128
HQ = 8
HKV = 2
GROUP = HQ // HKV
ROWS = GROUP * SQ
DH = 128
D = 1024
SCALE = 0.08838834764831843
CHUNK = 4096
M_SHIFT = 4.0


def kernel(x, Wq, Wo, K_ext, V_ext):
    x2 = x.reshape(SQ, D)
    skv = K_ext.shape[1]
    nc = skv // CHUNK

    def body(x_ref, wq_ref, wo_ref, k_hbm, v_hbm, out_ref,
             q_ref, acc_ref, l_ref, kv_buf, dma_sems,
             o_send, comm_o, comm_l,
             send_o, recv_o, send_l, recv_l, xout):
        g = pl.program_id(0)
        j = pl.program_id(1)
        step = g * nc + j
        slot = lax.rem(step, 2)
        my = lax.axis_index("i")

        def kv_copies(gg, jj, sl):
            return [
                pltpu.make_async_copy(
                    k_hbm.at[0, pl.ds(jj * CHUNK, CHUNK), gg, :],
                    kv_buf.at[sl, 0], dma_sems.at[sl, 0]),
                pltpu.make_async_copy(
                    v_hbm.at[0, pl.ds(jj * CHUNK, CHUNK), gg, :],
                    kv_buf.at[sl, 1], dma_sems.at[sl, 1]),
            ]

        def peer(k):
            return lax.rem(my + k, N_DEV)

        def bcast_ops(G):
            ops = []
            for k in range(1, N_DEV):
                ops.append(pltpu.make_async_remote_copy(
                    src_ref=o_send.at[G], dst_ref=comm_o.at[k - 1, G],
                    send_sem=send_o.at[G, k - 1], recv_sem=recv_o.at[G, k - 1],
                    device_id=(peer(k),), device_id_type=pl.DeviceIdType.MESH))
                ops.append(pltpu.make_async_remote_copy(
                    src_ref=l_ref.at[G], dst_ref=comm_l.at[k - 1, G],
                    send_sem=send_l.at[G, k - 1], recv_sem=recv_l.at[G, k - 1],
                    device_id=(peer(k),), device_id_type=pl.DeviceIdType.MESH))
            return ops

        @pl.when(step == 0)
        def _init():
            for op in kv_copies(0, 0, 0):
                op.start()
            q = jnp.dot(x_ref[...], wq_ref[...],
                        preferred_element_type=jnp.float32) * SCALE
            for gg in range(HKV):
                for i in range(GROUP):
                    h = gg * GROUP + i
                    q_ref[gg, i * SQ:(i + 1) * SQ, :] = q[:, h * DH:(h + 1) * DH]
            acc_ref[...] = jnp.zeros_like(acc_ref)
            l_ref[...] = jnp.zeros_like(l_ref)
            barrier = pltpu.get_barrier_semaphore()
            for k in range(1, N_DEV):
                pl.semaphore_signal(barrier, inc=1, device_id=(peer(k),),
                                    device_id_type=pl.DeviceIdType.MESH)
            pl.semaphore_wait(barrier, N_DEV - 1)

        @pl.when(step + 1 < HKV * nc)
        def _prefetch():
            nstep = step + 1
            for op in kv_copies(nstep // nc, lax.rem(nstep, nc), 1 - slot):
                op.start()

        for op in kv_copies(g, j, slot):
            op.wait()

        qg = q_ref[g]
        kg = kv_buf[slot, 0]
        vg = kv_buf[slot, 1]
        s = lax.dot_general(
            qg, kg, (((1,), (1,)), ((), ())),
            preferred_element_type=jnp.float32)
        p = jnp.exp(s - M_SHIFT)
        pv = lax.dot_general(
            p.astype(jnp.bfloat16), vg.astype(jnp.bfloat16),
            (((1,), (0,)), ((), ())),
            preferred_element_type=jnp.float32)
        lv = jnp.sum(p, axis=1, keepdims=True)
        acc_ref[g] = acc_ref[g] + pv
        l_ref[g] = l_ref[g] + lv

        for G in range(HKV):
            @pl.when((g == G) & (j == nc - 1))
            def _send(G=G):
                o_send[G] = acc_ref[G].astype(jnp.bfloat16)
                for op in bcast_ops(G):
                    op.start()

        @pl.when(step == HKV * nc - 1)
        def _finish():
            for G in range(HKV):
                for op in bcast_ops(G):
                    op.wait()
            for G in range(HKV):
                num = acc_ref[G]
                den = l_ref[G]
                for t in range(N_DEV - 1):
                    num = num + comm_o[t, G].astype(jnp.float32)
                    den = den + comm_l[t, G]
                res = num / den
                for i in range(GROUP):
                    h = G * GROUP + i
                    xout[:, h * DH:(h + 1) * DH] = res[i * SQ:(i + 1) * SQ, :]
            out_ref[...] = jnp.dot(xout[...], wo_ref[...],
                                   preferred_element_type=jnp.float32)

    out = pl.pallas_call(
        body,
        grid=(HKV, nc),
        in_specs=[
            pl.BlockSpec((SQ, D), lambda g, j: (0, 0)),
            pl.BlockSpec((D, D), lambda g, j: (0, 0)),
            pl.BlockSpec((D, D), lambda g, j: (0, 0)),
            pl.BlockSpec(memory_space=pl.ANY),
            pl.BlockSpec(memory_space=pl.ANY),
        ],
        out_specs=pl.BlockSpec((SQ, D), lambda g, j: (0, 0)),
        out_shape=jax.ShapeDtypeStruct((SQ, D), jnp.float32),
        scratch_shapes=[
            pltpu.VMEM((HKV, ROWS, DH), jnp.float32),
            pltpu.VMEM((HKV, ROWS, DH), jnp.float32),
            pltpu.VMEM((HKV, ROWS, 1), jnp.float32),
            pltpu.VMEM((2, 2, CHUNK, DH), jnp.float32),
            pltpu.SemaphoreType.DMA((2, 2)),
            pltpu.VMEM((HKV, ROWS, DH), jnp.bfloat16),
            pltpu.VMEM((N_DEV - 1, HKV, ROWS, DH), jnp.bfloat16),
            pltpu.VMEM((N_DEV - 1, HKV, ROWS, 1), jnp.float32),
            pltpu.SemaphoreType.DMA((HKV, N_DEV - 1)),
            pltpu.SemaphoreType.DMA((HKV, N_DEV - 1)),
            pltpu.SemaphoreType.DMA((HKV, N_DEV - 1)),
            pltpu.SemaphoreType.DMA((HKV, N_DEV - 1)),
            pltpu.VMEM((SQ, D), jnp.float32),
        ],
        compiler_params=pltpu.CompilerParams(collective_id=0),
    )(x2, Wq, Wo, K_ext, V_ext)
    return out.reshape(1, SQ, D)


# device time: 63683 ns/iter; 1.0340x vs baseline; 1.0340x over previous
import jax
import jax.numpy as jnp
from jax import lax
from jax.experimental import pallas as pl
from jax.experimental.pallas import tpu as pltpu

N_DEV = 4
SQ = 128
HQ = 8
HKV = 2
GROUP = HQ // HKV
ROWS = GROUP * SQ
DH = 128
D = 1024
SCALE = 0.08838834764831843
CHUNK = 4096
M_SHIFT = 4.0


def kernel(x, Wq, Wo, K_ext, V_ext):
    skv = K_ext.shape[1]
    nc = skv // CHUNK

    def body(x_ref, wq_ref, wo_ref, k_hbm, v_hbm, out_ref,
             q_ref, acc_ref, l_ref, kv_buf, dma_sems,
             o_send, comm_o, comm_l,
             send_o, recv_o, send_l, recv_l, xout):
        g = pl.program_id(0)
        j = pl.program_id(1)
        step = g * nc + j
        slot = lax.rem(step, 2)
        my = lax.axis_index("i")

        def kv_copies(gg, jj, sl):
            return [
                pltpu.make_async_copy(
                    k_hbm.at[0, pl.ds(jj * CHUNK, CHUNK), gg, :],
                    kv_buf.at[sl, 0], dma_sems.at[sl, 0]),
                pltpu.make_async_copy(
                    v_hbm.at[0, pl.ds(jj * CHUNK, CHUNK), gg, :],
                    kv_buf.at[sl, 1], dma_sems.at[sl, 1]),
            ]

        def peer(k):
            return lax.rem(my + k, N_DEV)

        def bcast_ops(G):
            ops = []
            for k in range(1, N_DEV):
                ops.append(pltpu.make_async_remote_copy(
                    src_ref=o_send.at[G], dst_ref=comm_o.at[k - 1, G],
                    send_sem=send_o.at[G, k - 1], recv_sem=recv_o.at[G, k - 1],
                    device_id=(peer(k),), device_id_type=pl.DeviceIdType.MESH))
                ops.append(pltpu.make_async_remote_copy(
                    src_ref=l_ref.at[G], dst_ref=comm_l.at[k - 1, G],
                    send_sem=send_l.at[G, k - 1], recv_sem=recv_l.at[G, k - 1],
                    device_id=(peer(k),), device_id_type=pl.DeviceIdType.MESH))
            return ops

        @pl.when(step == 0)
        def _init():
            for op in kv_copies(0, 0, 0):
                op.start()
            q = jnp.dot(x_ref[0], wq_ref[...],
                        preferred_element_type=jnp.float32) * SCALE
            for gg in range(HKV):
                for i in range(GROUP):
                    h = gg * GROUP + i
                    q_ref[gg, i * SQ:(i + 1) * SQ, :] = q[:, h * DH:(h + 1) * DH]
            acc_ref[...] = jnp.zeros_like(acc_ref)
            l_ref[...] = jnp.zeros_like(l_ref)
            barrier = pltpu.get_barrier_semaphore()
            for k in range(1, N_DEV):
                pl.semaphore_signal(barrier, inc=1, device_id=(peer(k),),
                                    device_id_type=pl.DeviceIdType.MESH)
            pl.semaphore_wait(barrier, N_DEV - 1)

        @pl.when(step + 1 < HKV * nc)
        def _prefetch():
            nstep = step + 1
            for op in kv_copies(nstep // nc, lax.rem(nstep, nc), 1 - slot):
                op.start()

        for op in kv_copies(g, j, slot):
            op.wait()

        qg = q_ref[g]
        kg = kv_buf[slot, 0]
        vg = kv_buf[slot, 1]
        s = lax.dot_general(
            qg, kg, (((1,), (1,)), ((), ())),
            preferred_element_type=jnp.float32)
        p = jnp.exp(s - M_SHIFT)
        pv = lax.dot_general(
            p, vg, (((1,), (0,)), ((), ())),
            preferred_element_type=jnp.float32)
        lv = jnp.sum(p, axis=1, keepdims=True)
        acc_ref[g] = acc_ref[g] + pv
        l_ref[g] = l_ref[g] + lv

        for G in range(HKV):
            @pl.when((g == G) & (j == nc - 1))
            def _send(G=G):
                o_send[G] = acc_ref[G].astype(jnp.bfloat16)
                for op in bcast_ops(G):
                    op.start()

        @pl.when(step == HKV * nc - 1)
        def _finish():
            for G in range(HKV):
                for op in bcast_ops(G):
                    op.wait()
            for G in range(HKV):
                num = acc_ref[G]
                den = l_ref[G]
                for t in range(N_DEV - 1):
                    num = num + comm_o[t, G].astype(jnp.float32)
                    den = den + comm_l[t, G]
                res = num / den
                for i in range(GROUP):
                    h = G * GROUP + i
                    xout[:, h * DH:(h + 1) * DH] = res[i * SQ:(i + 1) * SQ, :]
            out_ref[0] = jnp.dot(xout[...], wo_ref[...],
                                 preferred_element_type=jnp.float32)

    out = pl.pallas_call(
        body,
        grid=(HKV, nc),
        in_specs=[
            pl.BlockSpec((1, SQ, D), lambda g, j: (0, 0, 0)),
            pl.BlockSpec((D, D), lambda g, j: (0, 0)),
            pl.BlockSpec((D, D), lambda g, j: (0, 0)),
            pl.BlockSpec(memory_space=pl.ANY),
            pl.BlockSpec(memory_space=pl.ANY),
        ],
        out_specs=pl.BlockSpec((1, SQ, D), lambda g, j: (0, 0, 0)),
        out_shape=jax.ShapeDtypeStruct((1, SQ, D), jnp.float32),
        scratch_shapes=[
            pltpu.VMEM((HKV, ROWS, DH), jnp.float32),
            pltpu.VMEM((HKV, ROWS, DH), jnp.float32),
            pltpu.VMEM((HKV, ROWS, 1), jnp.float32),
            pltpu.VMEM((2, 2, CHUNK, DH), jnp.float32),
            pltpu.SemaphoreType.DMA((2, 2)),
            pltpu.VMEM((HKV, ROWS, DH), jnp.bfloat16),
            pltpu.VMEM((N_DEV - 1, HKV, ROWS, DH), jnp.bfloat16),
            pltpu.VMEM((N_DEV - 1, HKV, ROWS, 1), jnp.float32),
            pltpu.SemaphoreType.DMA((HKV, N_DEV - 1)),
            pltpu.SemaphoreType.DMA((HKV, N_DEV - 1)),
            pltpu.SemaphoreType.DMA((HKV, N_DEV - 1)),
            pltpu.SemaphoreType.DMA((HKV, N_DEV - 1)),
            pltpu.VMEM((SQ, D), jnp.float32),
        ],
        compiler_params=pltpu.CompilerParams(collective_id=0),
    )(x, Wq, Wo, K_ext, V_ext)
    return out
